# baseline (speedup 1.0000x reference)
import jax
import jax.numpy as jnp
from jax import lax
from jax.experimental import pallas as pl
from jax.experimental.pallas import tpu as pltpu
from jax.experimental.pallas import tpu_sc as plsc

NUM_FACTORS = 32
MU = 3.5
BATCH = 16384
NC = 2
NS = 16
L = 16
NW = NC * NS
BPW = BATCH // NW
CHUNK = 128
NCHUNK = BPW // CHUNK


def _sc_body(uidx_hbm, iidx_hbm, uwt_hbm, iwt_hbm, ub_hbm, ib_hbm, out_hbm,
             uidx_v, iidx_v, ubuf_v, ibuf_v, ub_v, ib_v, out_v, sem):
    c = lax.axis_index("c")
    s = lax.axis_index("s")
    wid = s * NC + c

    pltpu.sync_copy(uidx_hbm.at[wid], uidx_v)
    pltpu.sync_copy(iidx_hbm.at[wid], iidx_v)

    copies = []
    for j in range(NCHUNK):
        dst = pl.ds(j * CHUNK, CHUNK)
        copies.append(
            pltpu.async_copy(ub_hbm.at[uidx_v.at[j]], ub_v.at[dst], sem))
        copies.append(
            pltpu.async_copy(ib_hbm.at[iidx_v.at[j]], ib_v.at[dst], sem))
    for f in range(NUM_FACTORS):
        for j in range(NCHUNK):
            dst = pl.ds(j * CHUNK, CHUNK)
            copies.append(pltpu.async_copy(
                uwt_hbm.at[f].at[uidx_v.at[j]], ubuf_v.at[f, dst], sem))
            copies.append(pltpu.async_copy(
                iwt_hbm.at[f].at[iidx_v.at[j]], ibuf_v.at[f, dst], sem))
    for cp in copies:
        cp.wait()

    def group(g, carry):
        sl = pl.ds(g * L, L)
        acc = ub_v[sl] + ib_v[sl] + MU
        for f in range(NUM_FACTORS):
            acc = acc + ubuf_v[f, sl] * ibuf_v[f, sl]
        out_v[sl] = acc
        return carry

    lax.fori_loop(0, BPW // L, group, 0)

    pltpu.sync_copy(out_v, out_hbm.at[pl.ds(wid * BPW, BPW)])


@jax.jit
def _run(uidx3, iidx3, uw, iw, ubf, ibf):
    mesh = plsc.VectorSubcoreMesh(core_axis_name="c", subcore_axis_name="s")
    f = pl.kernel(
        _sc_body,
        mesh=mesh,
        compiler_params=pltpu.CompilerParams(use_tc_tiling_on_sc=False),
        out_type=jax.ShapeDtypeStruct((BATCH,), jnp.float32),
        scratch_types=[
            pltpu.VMEM((NCHUNK, CHUNK), jnp.int32),
            pltpu.VMEM((NCHUNK, CHUNK), jnp.int32),
            pltpu.VMEM((NUM_FACTORS, BPW), jnp.float32),
            pltpu.VMEM((NUM_FACTORS, BPW), jnp.float32),
            pltpu.VMEM((BPW,), jnp.float32),
            pltpu.VMEM((BPW,), jnp.float32),
            pltpu.VMEM((BPW,), jnp.float32),
            pltpu.SemaphoreType.DMA,
        ],
    )
    return f(uidx3, iidx3, uw, iw, ubf, ibf)


def kernel(user_idx, item_idx, embed_user_w, embed_item_w, user_bias_w, item_bias_w):
    uidx3 = user_idx.reshape(NW, NCHUNK, CHUNK)
    iidx3 = item_idx.reshape(NW, NCHUNK, CHUNK)
    one = (user_idx[0] * 0 + 1).astype(jnp.float32)
    uwt = embed_user_w.T * one
    iwt = embed_item_w.T * one
    ubf = user_bias_w.reshape(-1)
    ibf = item_bias_w.reshape(-1)
    return _run(uidx3, iidx3, uwt, iwt, ubf, ibf)


# speedup vs baseline: 5.8051x; 5.8051x over previous
import jax
import jax.numpy as jnp
from jax import lax
from jax.experimental import pallas as pl
from jax.experimental.pallas import tpu as pltpu
from jax.experimental.pallas import tpu_sc as plsc

NUM_FACTORS = 32
MU = 3.5
BATCH = 16384
NC = 2
NS = 16
L = 16
NW = NC * NS
BPW = BATCH // NW
CHUNK = 128
NCHUNK = BPW // CHUNK


def _sc_body(uidx_hbm, iidx_hbm, uw_hbm, iw_hbm, ub_hbm, ib_hbm, out_hbm,
             uidx_v, iidx_v, urows_v, irows_v, ub_v, ib_v, out_v, sem):
    c = lax.axis_index("c")
    s = lax.axis_index("s")
    wid = s * NC + c

    pltpu.sync_copy(uidx_hbm.at[wid], uidx_v)
    pltpu.sync_copy(iidx_hbm.at[wid], iidx_v)

    copies = []
    for j in range(NCHUNK):
        dst = pl.ds(j * CHUNK, CHUNK)
        copies.append(pltpu.async_copy(uw_hbm.at[uidx_v.at[j]], urows_v.at[dst], sem))
        copies.append(pltpu.async_copy(iw_hbm.at[iidx_v.at[j]], irows_v.at[dst], sem))
        copies.append(pltpu.async_copy(ub_hbm.at[uidx_v.at[j]], ub_v.at[dst], sem))
        copies.append(pltpu.async_copy(ib_hbm.at[iidx_v.at[j]], ib_v.at[dst], sem))
    for cp in copies:
        cp.wait()

    lane = lax.iota(jnp.int32, L)

    def group(g, carry):
        dots = jnp.zeros((L,), jnp.float32)
        for k in range(L):
            r = g * L + k
            u0 = urows_v[r, pl.ds(0, L)]
            u1 = urows_v[r, pl.ds(L, L)]
            i0 = irows_v[r, pl.ds(0, L)]
            i1 = irows_v[r, pl.ds(L, L)]
            v = u0 * i0 + u1 * i1
            h = v + lax.rev(v, (0,))
            s = (((h[0] + h[1]) + (h[2] + h[3]))
                 + ((h[4] + h[5]) + (h[6] + h[7])))
            dots = jnp.where(lane == k, s, dots)
        sl = pl.ds(g * L, L)
        out_v[sl] = dots + ub_v[sl] + ib_v[sl] + MU
        return carry

    lax.fori_loop(0, BPW // L, group, 0)

    pltpu.sync_copy(out_v, out_hbm.at[pl.ds(wid * BPW, BPW)])


@jax.jit
def _run(uidx3, iidx3, uw, iw, ubf, ibf):
    mesh = plsc.VectorSubcoreMesh(core_axis_name="c", subcore_axis_name="s")
    f = pl.kernel(
        _sc_body,
        mesh=mesh,
        compiler_params=pltpu.CompilerParams(use_tc_tiling_on_sc=False),
        out_type=jax.ShapeDtypeStruct((BATCH,), jnp.float32),
        scratch_types=[
            pltpu.VMEM((NCHUNK, CHUNK), jnp.int32),
            pltpu.VMEM((NCHUNK, CHUNK), jnp.int32),
            pltpu.VMEM((BPW, NUM_FACTORS), jnp.float32),
            pltpu.VMEM((BPW, NUM_FACTORS), jnp.float32),
            pltpu.VMEM((BPW,), jnp.float32),
            pltpu.VMEM((BPW,), jnp.float32),
            pltpu.VMEM((BPW,), jnp.float32),
            pltpu.SemaphoreType.DMA,
        ],
    )
    return f(uidx3, iidx3, uw, iw, ubf, ibf)


def kernel(user_idx, item_idx, embed_user_w, embed_item_w, user_bias_w, item_bias_w):
    uidx3 = user_idx.reshape(NW, NCHUNK, CHUNK)
    iidx3 = item_idx.reshape(NW, NCHUNK, CHUNK)
    ubf = user_bias_w.reshape(-1)
    ibf = item_bias_w.reshape(-1)
    return _run(uidx3, iidx3, embed_user_w, embed_item_w, ubf, ibf)
